# Initial kernel scaffold; baseline (speedup 1.0000x reference)
import functools

import jax
import jax.numpy as jnp
from jax.experimental import pallas as pl
from jax.experimental.pallas import tpu as pltpu

_VMEM_LIMIT = 32 * 1024 * 1024


def _conv_body(x_ref, w_ref, b_ref, o_ref, *, rb, W, cin, pool):
    cout = b_ref.shape[1]
    r0 = pl.program_id(1) * rb
    taps = []
    for dy in range(3):
        for dx in range(3):
            taps.append(x_ref[0, pl.ds(r0 + dy, rb), pl.ds(dx, W), :])
    a = jnp.concatenate(taps, axis=2).reshape(rb * W, 9 * cin)
    acc = jnp.dot(a, w_ref[...], preferred_element_type=jnp.float32)
    y = jnp.maximum(acc + b_ref[...], 0.0).astype(o_ref.dtype)
    y = y.reshape(rb, W, cout)
    if pool:
        y = y.reshape(rb // 2, 2, W // 2, 2, cout).max(axis=(1, 3))
    o_ref[...] = y[None]


def _conv_layer(x, w, b, *, rb, pool):
    N, H, W, cin = x.shape
    cout = w.shape[-1]
    xp = jnp.pad(x, ((0, 0), (1, 1), (1, 1), (0, 0)))
    wm = w.reshape(9 * cin, cout)
    ob, Ho, Wo = (rb // 2, H // 2, W // 2) if pool else (rb, H, W)
    return pl.pallas_call(
        functools.partial(_conv_body, rb=rb, W=W, cin=cin, pool=pool),
        out_shape=jax.ShapeDtypeStruct((N, Ho, Wo, cout), jnp.bfloat16),
        grid=(N, H // rb),
        in_specs=[
            pl.BlockSpec((1, H + 2, W + 2, cin), lambda n, r: (n, 0, 0, 0)),
            pl.BlockSpec((9 * cin, cout), lambda n, r: (0, 0)),
            pl.BlockSpec((1, cout), lambda n, r: (0, 0)),
        ],
        out_specs=pl.BlockSpec((1, ob, Wo, cout), lambda n, r: (n, r, 0, 0)),
        compiler_params=pltpu.CompilerParams(
            dimension_semantics=("parallel", "parallel"),
            vmem_limit_bytes=_VMEM_LIMIT),
    )(xp, wm, b)


def _fc0_body(a_ref, w_ref, b_ref, o_ref, acc_ref, *, nk):
    k = pl.program_id(1)
    p = jnp.dot(a_ref[...], w_ref[...], preferred_element_type=jnp.float32)

    @pl.when(k == 0)
    def _():
        acc_ref[...] = p

    @pl.when(k > 0)
    def _():
        acc_ref[...] = acc_ref[...] + p

    @pl.when(k == nk - 1)
    def _():
        o_ref[...] = jnp.maximum(acc_ref[...] + b_ref[...], 0.0
                                 ).astype(o_ref.dtype)


def _fc0(a, wt, bias, *, tk, tn):
    M, K = a.shape
    N = wt.shape[1]
    nk, nn = K // tk, N // tn
    return pl.pallas_call(
        functools.partial(_fc0_body, nk=nk),
        out_shape=jax.ShapeDtypeStruct((M, N), jnp.bfloat16),
        grid=(nn, nk),
        in_specs=[
            pl.BlockSpec((M, tk), lambda j, k: (0, k)),
            pl.BlockSpec((tk, tn), lambda j, k: (k, j)),
            pl.BlockSpec((1, tn), lambda j, k: (0, j)),
        ],
        out_specs=pl.BlockSpec((M, tn), lambda j, k: (0, j)),
        scratch_shapes=[pltpu.VMEM((M, tn), jnp.float32)],
        compiler_params=pltpu.CompilerParams(
            dimension_semantics=("parallel", "arbitrary"),
            vmem_limit_bytes=_VMEM_LIMIT),
    )(a, wt, bias.reshape(1, N).astype(jnp.float32))


def _head_body(a_ref, w1_ref, b1_ref, w2_ref, b2_ref, o_ref, acc_ref, *, nk):
    k = pl.program_id(0)
    p = jnp.dot(a_ref[...], w1_ref[...], preferred_element_type=jnp.float32)

    @pl.when(k == 0)
    def _():
        acc_ref[...] = p

    @pl.when(k > 0)
    def _():
        acc_ref[...] = acc_ref[...] + p

    @pl.when(k == nk - 1)
    def _():
        h = jnp.maximum(acc_ref[...] + b1_ref[...], 0.0).astype(jnp.bfloat16)
        z = jnp.dot(h, w2_ref[...], preferred_element_type=jnp.float32)
        z = z + b2_ref[...]
        m = jnp.max(z, axis=1, keepdims=True)
        e = jnp.exp(z - m)
        o_ref[...] = e / jnp.sum(e, axis=1, keepdims=True)


def _head(a, w1, b1, w2, b2, *, tk):
    M, K = a.shape
    N1 = w1.shape[1]
    K2, C = w2.shape
    nk = K // tk
    return pl.pallas_call(
        functools.partial(_head_body, nk=nk),
        out_shape=jax.ShapeDtypeStruct((M, C), jnp.float32),
        grid=(nk,),
        in_specs=[
            pl.BlockSpec((M, tk), lambda k: (0, k)),
            pl.BlockSpec((tk, N1), lambda k: (k, 0)),
            pl.BlockSpec((1, N1), lambda k: (0, 0)),
            pl.BlockSpec((K2, C), lambda k: (0, 0)),
            pl.BlockSpec((1, C), lambda k: (0, 0)),
        ],
        out_specs=pl.BlockSpec((M, C), lambda k: (0, 0)),
        scratch_shapes=[pltpu.VMEM((M, N1), jnp.float32)],
        compiler_params=pltpu.CompilerParams(
            dimension_semantics=("arbitrary",),
            vmem_limit_bytes=_VMEM_LIMIT),
    )(a, w1, b1.reshape(1, N1).astype(jnp.float32),
      w2, b2.reshape(1, C).astype(jnp.float32))


_RB = {224: 8, 112: 8, 56: 8, 28: 14, 14: 14}


def kernel(blk0_0_w, blk0_0_bias, blk0_1_w, blk0_1_bias,
           blk1_0_w, blk1_0_bias, blk1_1_w, blk1_1_bias,
           blk2_0_w, blk2_0_bias, blk2_1_w, blk2_1_bias,
           blk2_2_w, blk2_2_bias,
           blk3_0_w, blk3_0_bias, blk3_1_w, blk3_1_bias,
           blk3_2_w, blk3_2_bias,
           blk4_0_w, blk4_0_bias, blk4_1_w, blk4_1_bias,
           blk4_2_w, blk4_2_bias,
           fc0_wt, fc0_bias, fc1_wt, fc1_bias, fc2_wt, fc2_bias, x):
    blocks = [
        [(blk0_0_w, blk0_0_bias), (blk0_1_w, blk0_1_bias)],
        [(blk1_0_w, blk1_0_bias), (blk1_1_w, blk1_1_bias)],
        [(blk2_0_w, blk2_0_bias), (blk2_1_w, blk2_1_bias),
         (blk2_2_w, blk2_2_bias)],
        [(blk3_0_w, blk3_0_bias), (blk3_1_w, blk3_1_bias),
         (blk3_2_w, blk3_2_bias)],
        [(blk4_0_w, blk4_0_bias), (blk4_1_w, blk4_1_bias),
         (blk4_2_w, blk4_2_bias)],
    ]
    h = jnp.transpose(x, (0, 2, 3, 1)).astype(jnp.bfloat16)
    for layers in blocks:
        for li, (w, b) in enumerate(layers):
            rb = _RB[h.shape[1]]
            h = _conv_layer(h, w, b, rb=rb, pool=(li == len(layers) - 1))
    h = h.reshape(h.shape[0], -1)
    h = _fc0(h, fc0_wt, fc0_bias, tk=3584, tn=1024)
    return _head(h, fc1_wt, fc1_bias, fc2_wt, fc2_bias, tk=1024)


# speedup vs baseline: 1.3041x; 1.3041x over previous
import functools

import jax
import jax.numpy as jnp
from jax.experimental import pallas as pl
from jax.experimental.pallas import tpu as pltpu

_VMEM_LIMIT = 32 * 1024 * 1024


def _conv_body(x_ref, w_ref, b_ref, o_ref, *, rb, W, cin, pool):
    cout = b_ref.shape[1]
    r0 = pl.program_id(1) * rb
    acc = None
    for t in range(9):
        dy, dx = divmod(t, 3)
        a = x_ref[0, pl.ds(r0 + dy, rb), pl.ds(dx, W), :].reshape(rb * W, cin)
        d = jnp.dot(a, w_ref[t], preferred_element_type=jnp.float32)
        acc = d if acc is None else acc + d
    y = jnp.maximum(acc + b_ref[...], 0.0).astype(o_ref.dtype)
    y = y.reshape(rb, W, cout)
    if pool:
        y = y.reshape(rb // 2, 2, W // 2, 2, cout).max(axis=(1, 3))
    o_ref[...] = y[None]


def _conv_layer(x, w, b, *, rb, pool):
    N, H, W, cin = x.shape
    cout = w.shape[-1]
    xp = jnp.pad(x, ((0, 0), (1, 1), (1, 1), (0, 0)))
    wm = w.reshape(9, cin, cout)
    ob, Ho, Wo = (rb // 2, H // 2, W // 2) if pool else (rb, H, W)
    return pl.pallas_call(
        functools.partial(_conv_body, rb=rb, W=W, cin=cin, pool=pool),
        out_shape=jax.ShapeDtypeStruct((N, Ho, Wo, cout), jnp.bfloat16),
        grid=(N, H // rb),
        in_specs=[
            pl.BlockSpec((1, H + 2, W + 2, cin), lambda n, r: (n, 0, 0, 0)),
            pl.BlockSpec((9, cin, cout), lambda n, r: (0, 0, 0)),
            pl.BlockSpec((1, cout), lambda n, r: (0, 0)),
        ],
        out_specs=pl.BlockSpec((1, ob, Wo, cout), lambda n, r: (n, r, 0, 0)),
        compiler_params=pltpu.CompilerParams(
            dimension_semantics=("parallel", "parallel"),
            vmem_limit_bytes=_VMEM_LIMIT),
    )(xp, wm, b)


def _fc0_body(a_ref, w_ref, b_ref, o_ref, acc_ref, *, nk):
    k = pl.program_id(1)
    p = jnp.dot(a_ref[...], w_ref[...], preferred_element_type=jnp.float32)

    @pl.when(k == 0)
    def _():
        acc_ref[...] = p

    @pl.when(k > 0)
    def _():
        acc_ref[...] = acc_ref[...] + p

    @pl.when(k == nk - 1)
    def _():
        o_ref[...] = jnp.maximum(acc_ref[...] + b_ref[...], 0.0
                                 ).astype(o_ref.dtype)


def _fc0(a, wt, bias, *, tk, tn):
    M, K = a.shape
    N = wt.shape[1]
    nk, nn = K // tk, N // tn
    return pl.pallas_call(
        functools.partial(_fc0_body, nk=nk),
        out_shape=jax.ShapeDtypeStruct((M, N), jnp.bfloat16),
        grid=(nn, nk),
        in_specs=[
            pl.BlockSpec((M, tk), lambda j, k: (0, k)),
            pl.BlockSpec((tk, tn), lambda j, k: (k, j)),
            pl.BlockSpec((1, tn), lambda j, k: (0, j)),
        ],
        out_specs=pl.BlockSpec((M, tn), lambda j, k: (0, j)),
        scratch_shapes=[pltpu.VMEM((M, tn), jnp.float32)],
        compiler_params=pltpu.CompilerParams(
            dimension_semantics=("parallel", "arbitrary"),
            vmem_limit_bytes=_VMEM_LIMIT),
    )(a, wt, bias.reshape(1, N).astype(jnp.float32))


def _head_body(a_ref, w1_ref, b1_ref, w2_ref, b2_ref, o_ref, acc_ref, *, nk):
    k = pl.program_id(0)
    p = jnp.dot(a_ref[...], w1_ref[...], preferred_element_type=jnp.float32)

    @pl.when(k == 0)
    def _():
        acc_ref[...] = p

    @pl.when(k > 0)
    def _():
        acc_ref[...] = acc_ref[...] + p

    @pl.when(k == nk - 1)
    def _():
        h = jnp.maximum(acc_ref[...] + b1_ref[...], 0.0).astype(jnp.bfloat16)
        z = jnp.dot(h, w2_ref[...], preferred_element_type=jnp.float32)
        z = z + b2_ref[...]
        m = jnp.max(z, axis=1, keepdims=True)
        e = jnp.exp(z - m)
        o_ref[...] = e / jnp.sum(e, axis=1, keepdims=True)


def _head(a, w1, b1, w2, b2, *, tk):
    M, K = a.shape
    N1 = w1.shape[1]
    K2, C = w2.shape
    nk = K // tk
    return pl.pallas_call(
        functools.partial(_head_body, nk=nk),
        out_shape=jax.ShapeDtypeStruct((M, C), jnp.float32),
        grid=(nk,),
        in_specs=[
            pl.BlockSpec((M, tk), lambda k: (0, k)),
            pl.BlockSpec((tk, N1), lambda k: (k, 0)),
            pl.BlockSpec((1, N1), lambda k: (0, 0)),
            pl.BlockSpec((K2, C), lambda k: (0, 0)),
            pl.BlockSpec((1, C), lambda k: (0, 0)),
        ],
        out_specs=pl.BlockSpec((M, C), lambda k: (0, 0)),
        scratch_shapes=[pltpu.VMEM((M, N1), jnp.float32)],
        compiler_params=pltpu.CompilerParams(
            dimension_semantics=("arbitrary",),
            vmem_limit_bytes=_VMEM_LIMIT),
    )(a, w1, b1.reshape(1, N1).astype(jnp.float32),
      w2, b2.reshape(1, C).astype(jnp.float32))


_RB = {224: 8, 112: 8, 56: 8, 28: 14, 14: 14}


def kernel(blk0_0_w, blk0_0_bias, blk0_1_w, blk0_1_bias,
           blk1_0_w, blk1_0_bias, blk1_1_w, blk1_1_bias,
           blk2_0_w, blk2_0_bias, blk2_1_w, blk2_1_bias,
           blk2_2_w, blk2_2_bias,
           blk3_0_w, blk3_0_bias, blk3_1_w, blk3_1_bias,
           blk3_2_w, blk3_2_bias,
           blk4_0_w, blk4_0_bias, blk4_1_w, blk4_1_bias,
           blk4_2_w, blk4_2_bias,
           fc0_wt, fc0_bias, fc1_wt, fc1_bias, fc2_wt, fc2_bias, x):
    blocks = [
        [(blk0_0_w, blk0_0_bias), (blk0_1_w, blk0_1_bias)],
        [(blk1_0_w, blk1_0_bias), (blk1_1_w, blk1_1_bias)],
        [(blk2_0_w, blk2_0_bias), (blk2_1_w, blk2_1_bias),
         (blk2_2_w, blk2_2_bias)],
        [(blk3_0_w, blk3_0_bias), (blk3_1_w, blk3_1_bias),
         (blk3_2_w, blk3_2_bias)],
        [(blk4_0_w, blk4_0_bias), (blk4_1_w, blk4_1_bias),
         (blk4_2_w, blk4_2_bias)],
    ]
    h = jnp.transpose(x, (0, 2, 3, 1)).astype(jnp.bfloat16)
    for layers in blocks:
        for li, (w, b) in enumerate(layers):
            rb = _RB[h.shape[1]]
            h = _conv_layer(h, w, b, rb=rb, pool=(li == len(layers) - 1))
    h = h.reshape(h.shape[0], -1)
    h = _fc0(h, fc0_wt, fc0_bias, tk=3584, tn=1024)
    return _head(h, fc1_wt, fc1_bias, fc2_wt, fc2_bias, tk=1024)


# speedup vs baseline: 1.4830x; 1.1372x over previous
import functools

import jax
import jax.numpy as jnp
from jax.experimental import pallas as pl
from jax.experimental.pallas import tpu as pltpu

_VMEM_LIMIT = 56 * 1024 * 1024


def _conv_body(x_ref, w_ref, b_ref, o_ref, *, rb, W, cin, pool):
    cout = b_ref.shape[1]
    r0 = pl.program_id(1) * rb
    cols = [x_ref[0, pl.ds(r0, rb + 2), pl.ds(dx, W), :] for dx in range(3)]
    acc = None
    for t in range(9):
        dy, dx = divmod(t, 3)
        a = cols[dx][dy:dy + rb].reshape(rb * W, cin)
        d = jnp.dot(a, w_ref[t], preferred_element_type=jnp.float32)
        acc = d if acc is None else acc + d
    y = jnp.maximum(acc + b_ref[...], 0.0)
    if pool:
        v = jnp.max(y.reshape(rb // 2, 2, W, cout), axis=1)
        h = jnp.max(v.reshape(rb // 2, W // 2, 2, cout), axis=2)
        o_ref[...] = h.astype(o_ref.dtype)[None]
    else:
        o_ref[...] = y.astype(o_ref.dtype).reshape(1, rb, W, cout)


def _conv_layer(x, w, b, *, rb, pool):
    N, H, W, cin = x.shape
    cout = w.shape[-1]
    xp = jnp.pad(x, ((0, 0), (1, 1), (1, 1), (0, 0)))
    wm = w.reshape(9, cin, cout)
    ob, Ho, Wo = (rb // 2, H // 2, W // 2) if pool else (rb, H, W)
    return pl.pallas_call(
        functools.partial(_conv_body, rb=rb, W=W, cin=cin, pool=pool),
        out_shape=jax.ShapeDtypeStruct((N, Ho, Wo, cout), jnp.bfloat16),
        grid=(N, H // rb),
        in_specs=[
            pl.BlockSpec((1, H + 2, W + 2, cin), lambda n, r: (n, 0, 0, 0)),
            pl.BlockSpec((9, cin, cout), lambda n, r: (0, 0, 0)),
            pl.BlockSpec((1, cout), lambda n, r: (0, 0)),
        ],
        out_specs=pl.BlockSpec((1, ob, Wo, cout), lambda n, r: (n, r, 0, 0)),
        compiler_params=pltpu.CompilerParams(
            dimension_semantics=("parallel", "parallel"),
            vmem_limit_bytes=_VMEM_LIMIT),
    )(xp, wm, b)


def _fc0_body(a_ref, w_ref, b_ref, o_ref, acc_ref, *, nk):
    k = pl.program_id(1)
    p = jnp.dot(a_ref[...], w_ref[...], preferred_element_type=jnp.float32)

    @pl.when(k == 0)
    def _():
        acc_ref[...] = p

    @pl.when(k > 0)
    def _():
        acc_ref[...] = acc_ref[...] + p

    @pl.when(k == nk - 1)
    def _():
        o_ref[...] = jnp.maximum(acc_ref[...] + b_ref[...], 0.0
                                 ).astype(o_ref.dtype)


def _fc0(a, wt, bias, *, tk, tn):
    M, K = a.shape
    N = wt.shape[1]
    nk, nn = K // tk, N // tn
    return pl.pallas_call(
        functools.partial(_fc0_body, nk=nk),
        out_shape=jax.ShapeDtypeStruct((M, N), jnp.bfloat16),
        grid=(nn, nk),
        in_specs=[
            pl.BlockSpec((M, tk), lambda j, k: (0, k)),
            pl.BlockSpec((tk, tn), lambda j, k: (k, j)),
            pl.BlockSpec((1, tn), lambda j, k: (0, j)),
        ],
        out_specs=pl.BlockSpec((M, tn), lambda j, k: (0, j)),
        scratch_shapes=[pltpu.VMEM((M, tn), jnp.float32)],
        compiler_params=pltpu.CompilerParams(
            dimension_semantics=("parallel", "arbitrary"),
            vmem_limit_bytes=_VMEM_LIMIT),
    )(a, wt, bias.reshape(1, N).astype(jnp.float32))


def _head_body(a_ref, w1_ref, b1_ref, w2_ref, b2_ref, o_ref, acc_ref, *, nk):
    k = pl.program_id(0)
    p = jnp.dot(a_ref[...], w1_ref[...], preferred_element_type=jnp.float32)

    @pl.when(k == 0)
    def _():
        acc_ref[...] = p

    @pl.when(k > 0)
    def _():
        acc_ref[...] = acc_ref[...] + p

    @pl.when(k == nk - 1)
    def _():
        h = jnp.maximum(acc_ref[...] + b1_ref[...], 0.0).astype(jnp.bfloat16)
        z = jnp.dot(h, w2_ref[...], preferred_element_type=jnp.float32)
        z = z + b2_ref[...]
        m = jnp.max(z, axis=1, keepdims=True)
        e = jnp.exp(z - m)
        o_ref[...] = e / jnp.sum(e, axis=1, keepdims=True)


def _head(a, w1, b1, w2, b2, *, tk):
    M, K = a.shape
    N1 = w1.shape[1]
    K2, C = w2.shape
    nk = K // tk
    return pl.pallas_call(
        functools.partial(_head_body, nk=nk),
        out_shape=jax.ShapeDtypeStruct((M, C), jnp.float32),
        grid=(nk,),
        in_specs=[
            pl.BlockSpec((M, tk), lambda k: (0, k)),
            pl.BlockSpec((tk, N1), lambda k: (k, 0)),
            pl.BlockSpec((1, N1), lambda k: (0, 0)),
            pl.BlockSpec((K2, C), lambda k: (0, 0)),
            pl.BlockSpec((1, C), lambda k: (0, 0)),
        ],
        out_specs=pl.BlockSpec((M, C), lambda k: (0, 0)),
        scratch_shapes=[pltpu.VMEM((M, N1), jnp.float32)],
        compiler_params=pltpu.CompilerParams(
            dimension_semantics=("arbitrary",),
            vmem_limit_bytes=_VMEM_LIMIT),
    )(a, w1, b1.reshape(1, N1).astype(jnp.float32),
      w2, b2.reshape(1, C).astype(jnp.float32))


_RB = {224: 8, 112: 8, 56: 8, 28: 14, 14: 14}


def kernel(blk0_0_w, blk0_0_bias, blk0_1_w, blk0_1_bias,
           blk1_0_w, blk1_0_bias, blk1_1_w, blk1_1_bias,
           blk2_0_w, blk2_0_bias, blk2_1_w, blk2_1_bias,
           blk2_2_w, blk2_2_bias,
           blk3_0_w, blk3_0_bias, blk3_1_w, blk3_1_bias,
           blk3_2_w, blk3_2_bias,
           blk4_0_w, blk4_0_bias, blk4_1_w, blk4_1_bias,
           blk4_2_w, blk4_2_bias,
           fc0_wt, fc0_bias, fc1_wt, fc1_bias, fc2_wt, fc2_bias, x):
    blocks = [
        [(blk0_0_w, blk0_0_bias), (blk0_1_w, blk0_1_bias)],
        [(blk1_0_w, blk1_0_bias), (blk1_1_w, blk1_1_bias)],
        [(blk2_0_w, blk2_0_bias), (blk2_1_w, blk2_1_bias),
         (blk2_2_w, blk2_2_bias)],
        [(blk3_0_w, blk3_0_bias), (blk3_1_w, blk3_1_bias),
         (blk3_2_w, blk3_2_bias)],
        [(blk4_0_w, blk4_0_bias), (blk4_1_w, blk4_1_bias),
         (blk4_2_w, blk4_2_bias)],
    ]
    h = jnp.transpose(x, (0, 2, 3, 1)).astype(jnp.bfloat16)
    for layers in blocks:
        for li, (w, b) in enumerate(layers):
            rb = _RB[h.shape[1]]
            h = _conv_layer(h, w, b, rb=rb, pool=(li == len(layers) - 1))
    h = h.reshape(h.shape[0], -1)
    h = _fc0(h, fc0_wt, fc0_bias, tk=3584, tn=1024)
    return _head(h, fc1_wt, fc1_bias, fc2_wt, fc2_bias, tk=1024)


# speedup vs baseline: 1.5658x; 1.0559x over previous
import functools

import jax
import jax.numpy as jnp
from jax.experimental import pallas as pl
from jax.experimental.pallas import tpu as pltpu

_VMEM_LIMIT = 56 * 1024 * 1024


def _conv_body(x_ref, w_ref, b_ref, o_ref, *, rb, W, cin, pool):
    cout = b_ref.shape[1]
    r0 = pl.program_id(1) * rb
    cols = [x_ref[0, pl.ds(r0, rb + 2), pl.ds(dx, W), :] for dx in range(3)]
    acc = None
    for t in range(9):
        dy, dx = divmod(t, 3)
        a = cols[dx][dy:dy + rb].reshape(rb * W, cin)
        d = jnp.dot(a, w_ref[t], preferred_element_type=jnp.float32)
        acc = d if acc is None else acc + d
    y = jnp.maximum(acc + b_ref[...], 0.0)
    if pool:
        v = jnp.max(y.reshape(rb // 2, 2, W, cout), axis=1)
        h = jnp.max(v.reshape(rb // 2, W // 2, 2, cout), axis=2)
        o_ref[...] = h.astype(o_ref.dtype)[None]
    else:
        o_ref[...] = y.astype(o_ref.dtype).reshape(1, rb, W, cout)


def _conv_layer(x, w, b, *, rb, pool):
    N, H, W, cin = x.shape
    cout = w.shape[-1]
    xp = jnp.pad(x, ((0, 0), (1, 1), (1, 1), (0, 0)))
    wm = w.reshape(9, cin, cout)
    ob, Ho, Wo = (rb // 2, H // 2, W // 2) if pool else (rb, H, W)
    return pl.pallas_call(
        functools.partial(_conv_body, rb=rb, W=W, cin=cin, pool=pool),
        out_shape=jax.ShapeDtypeStruct((N, Ho, Wo, cout), jnp.bfloat16),
        grid=(N, H // rb),
        in_specs=[
            pl.BlockSpec((1, H + 2, W + 2, cin), lambda n, r: (n, 0, 0, 0)),
            pl.BlockSpec((9, cin, cout), lambda n, r: (0, 0, 0)),
            pl.BlockSpec((1, cout), lambda n, r: (0, 0)),
        ],
        out_specs=pl.BlockSpec((1, ob, Wo, cout), lambda n, r: (n, r, 0, 0)),
        compiler_params=pltpu.CompilerParams(
            dimension_semantics=("parallel", "parallel"),
            vmem_limit_bytes=_VMEM_LIMIT),
    )(xp, wm, b)


def _fc0_body(a_ref, w_ref, b_ref, o_ref, acc_ref, *, nk):
    k = pl.program_id(1)
    p = jnp.dot(a_ref[...], w_ref[...], preferred_element_type=jnp.float32)

    @pl.when(k == 0)
    def _():
        acc_ref[...] = p

    @pl.when(k > 0)
    def _():
        acc_ref[...] = acc_ref[...] + p

    @pl.when(k == nk - 1)
    def _():
        o_ref[...] = jnp.maximum(acc_ref[...] + b_ref[...], 0.0
                                 ).astype(o_ref.dtype)


def _fc0(a, wt, bias, *, tk, tn):
    M, K = a.shape
    N = wt.shape[1]
    nk, nn = K // tk, N // tn
    return pl.pallas_call(
        functools.partial(_fc0_body, nk=nk),
        out_shape=jax.ShapeDtypeStruct((M, N), jnp.bfloat16),
        grid=(nn, nk),
        in_specs=[
            pl.BlockSpec((M, tk), lambda j, k: (0, k)),
            pl.BlockSpec((tk, tn), lambda j, k: (k, j)),
            pl.BlockSpec((1, tn), lambda j, k: (0, j)),
        ],
        out_specs=pl.BlockSpec((M, tn), lambda j, k: (0, j)),
        scratch_shapes=[pltpu.VMEM((M, tn), jnp.float32)],
        compiler_params=pltpu.CompilerParams(
            dimension_semantics=("parallel", "arbitrary"),
            vmem_limit_bytes=_VMEM_LIMIT),
    )(a, wt, bias.reshape(1, N).astype(jnp.float32))


def _head_body(a_ref, w1_ref, b1_ref, w2_ref, b2_ref, o_ref, acc_ref, *, nk):
    k = pl.program_id(0)
    p = jnp.dot(a_ref[...], w1_ref[...], preferred_element_type=jnp.float32)

    @pl.when(k == 0)
    def _():
        acc_ref[...] = p

    @pl.when(k > 0)
    def _():
        acc_ref[...] = acc_ref[...] + p

    @pl.when(k == nk - 1)
    def _():
        h = jnp.maximum(acc_ref[...] + b1_ref[...], 0.0).astype(jnp.bfloat16)
        z = jnp.dot(h, w2_ref[...], preferred_element_type=jnp.float32)
        z = z + b2_ref[...]
        m = jnp.max(z, axis=1, keepdims=True)
        e = jnp.exp(z - m)
        o_ref[...] = e / jnp.sum(e, axis=1, keepdims=True)


def _head(a, w1, b1, w2, b2, *, tk):
    M, K = a.shape
    N1 = w1.shape[1]
    K2, C = w2.shape
    nk = K // tk
    return pl.pallas_call(
        functools.partial(_head_body, nk=nk),
        out_shape=jax.ShapeDtypeStruct((M, C), jnp.float32),
        grid=(nk,),
        in_specs=[
            pl.BlockSpec((M, tk), lambda k: (0, k)),
            pl.BlockSpec((tk, N1), lambda k: (k, 0)),
            pl.BlockSpec((1, N1), lambda k: (0, 0)),
            pl.BlockSpec((K2, C), lambda k: (0, 0)),
            pl.BlockSpec((1, C), lambda k: (0, 0)),
        ],
        out_specs=pl.BlockSpec((M, C), lambda k: (0, 0)),
        scratch_shapes=[pltpu.VMEM((M, N1), jnp.float32)],
        compiler_params=pltpu.CompilerParams(
            dimension_semantics=("arbitrary",),
            vmem_limit_bytes=_VMEM_LIMIT),
    )(a, w1, b1.reshape(1, N1).astype(jnp.float32),
      w2, b2.reshape(1, C).astype(jnp.float32))


_RB = {224: 16, 112: 14, 56: 14, 28: 14, 14: 14}


def kernel(blk0_0_w, blk0_0_bias, blk0_1_w, blk0_1_bias,
           blk1_0_w, blk1_0_bias, blk1_1_w, blk1_1_bias,
           blk2_0_w, blk2_0_bias, blk2_1_w, blk2_1_bias,
           blk2_2_w, blk2_2_bias,
           blk3_0_w, blk3_0_bias, blk3_1_w, blk3_1_bias,
           blk3_2_w, blk3_2_bias,
           blk4_0_w, blk4_0_bias, blk4_1_w, blk4_1_bias,
           blk4_2_w, blk4_2_bias,
           fc0_wt, fc0_bias, fc1_wt, fc1_bias, fc2_wt, fc2_bias, x):
    blocks = [
        [(blk0_0_w, blk0_0_bias), (blk0_1_w, blk0_1_bias)],
        [(blk1_0_w, blk1_0_bias), (blk1_1_w, blk1_1_bias)],
        [(blk2_0_w, blk2_0_bias), (blk2_1_w, blk2_1_bias),
         (blk2_2_w, blk2_2_bias)],
        [(blk3_0_w, blk3_0_bias), (blk3_1_w, blk3_1_bias),
         (blk3_2_w, blk3_2_bias)],
        [(blk4_0_w, blk4_0_bias), (blk4_1_w, blk4_1_bias),
         (blk4_2_w, blk4_2_bias)],
    ]
    h = jnp.transpose(x, (0, 2, 3, 1)).astype(jnp.bfloat16)
    for layers in blocks:
        for li, (w, b) in enumerate(layers):
            rb = _RB[h.shape[1]]
            h = _conv_layer(h, w, b, rb=rb, pool=(li == len(layers) - 1))
    h = h.reshape(h.shape[0], -1)
    h = _fc0(h, fc0_wt, fc0_bias, tk=3584, tn=1024)
    return _head(h, fc1_wt, fc1_bias, fc2_wt, fc2_bias, tk=1024)


# speedup vs baseline: 1.6409x; 1.0480x over previous
import functools

import jax
import jax.numpy as jnp
from jax.experimental import pallas as pl
from jax.experimental.pallas import tpu as pltpu

_VMEM_LIMIT = 56 * 1024 * 1024


def _conv_body(x_ref, w_ref, b_ref, o_ref, *, rb, W, cin, pool):
    cout = b_ref.shape[1]
    r0 = pl.program_id(1) * rb
    cols = [x_ref[0, pl.ds(r0, rb + 2), pl.ds(dx, W), :] for dx in range(3)]
    acc = None
    for t in range(9):
        dy, dx = divmod(t, 3)
        a = cols[dx][dy:dy + rb].reshape(rb * W, cin)
        d = jnp.dot(a, w_ref[t], preferred_element_type=jnp.float32)
        acc = d if acc is None else acc + d
    y = jnp.maximum(acc + b_ref[...], 0.0)
    if pool:
        v = jnp.max(y.reshape(rb // 2, 2, W, cout), axis=1)
        h = jnp.max(v.reshape(rb // 2, W // 2, 2, cout), axis=2)
        o_ref[...] = h.astype(o_ref.dtype)[None]
    else:
        o_ref[...] = y.astype(o_ref.dtype).reshape(1, rb, W, cout)


def _conv_layer(x, w, b, *, rb, pool):
    N, H, W, cin = x.shape
    cout = w.shape[-1]
    xp = jnp.pad(x, ((0, 0), (1, 1), (1, 1), (0, 0)))
    wm = w.reshape(9, cin, cout)
    ob, Ho, Wo = (rb // 2, H // 2, W // 2) if pool else (rb, H, W)
    return pl.pallas_call(
        functools.partial(_conv_body, rb=rb, W=W, cin=cin, pool=pool),
        out_shape=jax.ShapeDtypeStruct((N, Ho, Wo, cout), jnp.bfloat16),
        grid=(N, H // rb),
        in_specs=[
            pl.BlockSpec((1, H + 2, W + 2, cin), lambda n, r: (n, 0, 0, 0)),
            pl.BlockSpec((9, cin, cout), lambda n, r: (0, 0, 0)),
            pl.BlockSpec((1, cout), lambda n, r: (0, 0)),
        ],
        out_specs=pl.BlockSpec((1, ob, Wo, cout), lambda n, r: (n, r, 0, 0)),
        compiler_params=pltpu.CompilerParams(
            dimension_semantics=("parallel", "parallel"),
            vmem_limit_bytes=_VMEM_LIMIT),
    )(xp, wm, b)


def _fc0_body(a_ref, w_ref, b_ref, o_ref, acc_ref, *, nk):
    k = pl.program_id(1)
    p = jnp.dot(a_ref[...], w_ref[...], preferred_element_type=jnp.float32)

    @pl.when(k == 0)
    def _():
        acc_ref[...] = p

    @pl.when(k > 0)
    def _():
        acc_ref[...] = acc_ref[...] + p

    @pl.when(k == nk - 1)
    def _():
        o_ref[...] = jnp.maximum(acc_ref[...] + b_ref[...], 0.0
                                 ).astype(o_ref.dtype)


def _fc0(a, wt, bias, *, tk, tn):
    M, K = a.shape
    N = wt.shape[1]
    nk, nn = K // tk, N // tn
    return pl.pallas_call(
        functools.partial(_fc0_body, nk=nk),
        out_shape=jax.ShapeDtypeStruct((M, N), jnp.bfloat16),
        grid=(nn, nk),
        in_specs=[
            pl.BlockSpec((M, tk), lambda j, k: (0, k)),
            pl.BlockSpec((tk, tn), lambda j, k: (k, j)),
            pl.BlockSpec((1, tn), lambda j, k: (0, j)),
        ],
        out_specs=pl.BlockSpec((M, tn), lambda j, k: (0, j)),
        scratch_shapes=[pltpu.VMEM((M, tn), jnp.float32)],
        compiler_params=pltpu.CompilerParams(
            dimension_semantics=("parallel", "arbitrary"),
            vmem_limit_bytes=_VMEM_LIMIT),
    )(a, wt, bias.reshape(1, N).astype(jnp.float32))


def _head_body(a_ref, w1_ref, b1_ref, w2_ref, b2_ref, o_ref, acc_ref, *, nk):
    k = pl.program_id(0)
    p = jnp.dot(a_ref[...], w1_ref[...], preferred_element_type=jnp.float32)

    @pl.when(k == 0)
    def _():
        acc_ref[...] = p

    @pl.when(k > 0)
    def _():
        acc_ref[...] = acc_ref[...] + p

    @pl.when(k == nk - 1)
    def _():
        h = jnp.maximum(acc_ref[...] + b1_ref[...], 0.0).astype(jnp.bfloat16)
        z = jnp.dot(h, w2_ref[...], preferred_element_type=jnp.float32)
        z = z + b2_ref[...]
        m = jnp.max(z, axis=1, keepdims=True)
        e = jnp.exp(z - m)
        o_ref[...] = e / jnp.sum(e, axis=1, keepdims=True)


def _head(a, w1, b1, w2, b2, *, tk):
    M, K = a.shape
    N1 = w1.shape[1]
    K2, C = w2.shape
    nk = K // tk
    return pl.pallas_call(
        functools.partial(_head_body, nk=nk),
        out_shape=jax.ShapeDtypeStruct((M, C), jnp.float32),
        grid=(nk,),
        in_specs=[
            pl.BlockSpec((M, tk), lambda k: (0, k)),
            pl.BlockSpec((tk, N1), lambda k: (k, 0)),
            pl.BlockSpec((1, N1), lambda k: (0, 0)),
            pl.BlockSpec((K2, C), lambda k: (0, 0)),
            pl.BlockSpec((1, C), lambda k: (0, 0)),
        ],
        out_specs=pl.BlockSpec((M, C), lambda k: (0, 0)),
        scratch_shapes=[pltpu.VMEM((M, N1), jnp.float32)],
        compiler_params=pltpu.CompilerParams(
            dimension_semantics=("arbitrary",),
            vmem_limit_bytes=_VMEM_LIMIT),
    )(a, w1, b1.reshape(1, N1).astype(jnp.float32),
      w2, b2.reshape(1, C).astype(jnp.float32))


_RB = {224: 32, 112: 28, 56: 28, 28: 28, 14: 14}


def kernel(blk0_0_w, blk0_0_bias, blk0_1_w, blk0_1_bias,
           blk1_0_w, blk1_0_bias, blk1_1_w, blk1_1_bias,
           blk2_0_w, blk2_0_bias, blk2_1_w, blk2_1_bias,
           blk2_2_w, blk2_2_bias,
           blk3_0_w, blk3_0_bias, blk3_1_w, blk3_1_bias,
           blk3_2_w, blk3_2_bias,
           blk4_0_w, blk4_0_bias, blk4_1_w, blk4_1_bias,
           blk4_2_w, blk4_2_bias,
           fc0_wt, fc0_bias, fc1_wt, fc1_bias, fc2_wt, fc2_bias, x):
    blocks = [
        [(blk0_0_w, blk0_0_bias), (blk0_1_w, blk0_1_bias)],
        [(blk1_0_w, blk1_0_bias), (blk1_1_w, blk1_1_bias)],
        [(blk2_0_w, blk2_0_bias), (blk2_1_w, blk2_1_bias),
         (blk2_2_w, blk2_2_bias)],
        [(blk3_0_w, blk3_0_bias), (blk3_1_w, blk3_1_bias),
         (blk3_2_w, blk3_2_bias)],
        [(blk4_0_w, blk4_0_bias), (blk4_1_w, blk4_1_bias),
         (blk4_2_w, blk4_2_bias)],
    ]
    h = jnp.transpose(x, (0, 2, 3, 1)).astype(jnp.bfloat16)
    for layers in blocks:
        for li, (w, b) in enumerate(layers):
            rb = _RB[h.shape[1]]
            h = _conv_layer(h, w, b, rb=rb, pool=(li == len(layers) - 1))
    h = h.reshape(h.shape[0], -1)
    h = _fc0(h, fc0_wt, fc0_bias, tk=3584, tn=1024)
    return _head(h, fc1_wt, fc1_bias, fc2_wt, fc2_bias, tk=1024)
